# Initial kernel scaffold; baseline (speedup 1.0000x reference)
import functools

import jax
import jax.numpy as jnp
from jax import lax
from jax.experimental import pallas as pl
from jax.experimental.pallas import tpu as pltpu
from jax.experimental.pallas import tpu_sc as plsc

N = 10000
M = 32
F = 128
EDGE = 16
NE = N * M
B = 100
BM = B * M


def _sc_gather(table, idx):
    info = plsc.get_sparse_core_info()
    nc, ns = info.num_cores, info.num_subcores
    nw = nc * ns
    pw = NE // nw
    ch = 80
    nch = pw // ch

    mesh = plsc.VectorSubcoreMesh(core_axis_name="c", subcore_axis_name="s")

    @functools.partial(
        pl.kernel,
        out_type=jax.ShapeDtypeStruct((NE, 2 * F), jnp.float32),
        mesh=mesh,
        scratch_types=[
            pltpu.VMEM((ch,), jnp.int32),
            pltpu.VMEM((ch, 2 * F), jnp.float32),
            pltpu.SemaphoreType.DMA,
        ],
    )
    def gather_k(table_hbm, idx_hbm, out_hbm, idx_v, rows_v, sem):
        wid = lax.axis_index("s") * nc + lax.axis_index("c")
        base = wid * pw

        def body(c, carry):
            off = base + c * ch
            pltpu.sync_copy(idx_hbm.at[pl.ds(off, ch)], idx_v)
            pltpu.async_copy(table_hbm.at[idx_v], rows_v, sem).wait()
            pltpu.sync_copy(rows_v, out_hbm.at[pl.ds(off, ch)])
            return carry

        lax.fori_loop(0, nch, body, 0)

    return gather_k(table, idx)


def _softplus(x):
    return jnp.maximum(x, 0.0) + jnp.log1p(jnp.exp(-jnp.abs(x)))


def _sigmoid(x):
    return jax.nn.sigmoid(x)


def _dot(a, b):
    return lax.dot_general(a, b, (((1,), (0,)), ((), ())),
                           preferred_element_type=jnp.float32)


def _bc(x):
    return jnp.broadcast_to(x[:, None, :], (B, M, F)).reshape(BM, F)


def _tc_body(g_ref, nbr_ref, e_ref, o_ref,
             em1a, em1b, em1c, em1d, bem1,
             weg, beg, wem2, bem2,
             woej, boej, woei, boei, om1a, om1b,
             ogha, oghb, oghc, oghd, bogh, wog, bog,
             eout_ref, oout_ref):
    e_i = e_ref[...]
    o_i = o_ref[...]
    g = g_ref[...]
    ej = g[:, :F]
    oj = g[:, F:]
    nbr = nbr_ref[...]

    cross = _bc(o_i) * oj

    pre_e = (_bc(_dot(e_i, em1a[...])) + _dot(ej, em1b[...])
             + _dot(nbr, em1c[...]) + _dot(cross, em1d[...]) + bem1[...])
    h = _softplus(pre_e)
    gate = _sigmoid(_dot(h, weg[...]) + beg[...])
    msg = _softplus(_dot(h, wem2[...]) + bem2[...])
    even_agg = jnp.sum((gate * msg).reshape(B, M, F), axis=1)

    odd_ie = _bc(o_i) * (_dot(ej, woej[...]) + boej[...])
    odd_ei = _bc(_dot(e_i, woei[...]) + boei[...]) * oj
    odd_val = jnp.tanh(_dot(odd_ie, om1a[...]) + _dot(odd_ei, om1b[...]))

    pre_g = (_bc(_dot(e_i, ogha[...])) + _dot(ej, oghb[...])
             + _dot(nbr, oghc[...]) + _dot(cross, oghd[...]) + bogh[...])
    ogate = _sigmoid(_dot(_softplus(pre_g), wog[...]) + bog[...])
    odd_agg = jnp.sum((ogate * odd_val).reshape(B, M, F), axis=1)

    eout_ref[...] = e_i + even_agg
    oout_ref[...] = o_i + odd_agg


def _full(shape):
    return pl.BlockSpec(shape, lambda i: (0, 0))


def _tc_compute(gathered, nbr2, even_node, odd_node, weights):
    in_specs = [
        pl.BlockSpec((BM, 2 * F), lambda i: (i, 0)),
        pl.BlockSpec((BM, EDGE), lambda i: (i, 0)),
        pl.BlockSpec((B, F), lambda i: (i, 0)),
        pl.BlockSpec((B, F), lambda i: (i, 0)),
    ] + [_full(w.shape) for w in weights]
    out = pl.pallas_call(
        _tc_body,
        grid=(N // B,),
        in_specs=in_specs,
        out_specs=[pl.BlockSpec((B, F), lambda i: (i, 0))] * 2,
        out_shape=[jax.ShapeDtypeStruct((N, F), jnp.float32)] * 2,
    )(gathered, nbr2, even_node, odd_node, *weights)
    return tuple(out)


def kernel(even_node, odd_node, nbr_fea, nbr_fea_idx,
           W_em1, b_em1, W_eg, b_eg, W_em2, b_em2,
           W_oej, b_oej, W_oei, b_oei, W_om1,
           W_ogh, b_ogh, W_og, b_og):
    table = jnp.concatenate([even_node, odd_node], axis=1)
    idx = nbr_fea_idx.reshape(-1).astype(jnp.int32)
    gathered = _sc_gather(table, idx)
    nbr2 = nbr_fea.reshape(NE, EDGE)

    r = lambda b: b.reshape(1, F)
    weights = (
        W_em1[:F], W_em1[F:2 * F], W_em1[2 * F:2 * F + EDGE], W_em1[2 * F + EDGE:], r(b_em1),
        W_eg, r(b_eg), W_em2, r(b_em2),
        W_oej, r(b_oej), W_oei, r(b_oei), W_om1[:F], W_om1[F:],
        W_ogh[:F], W_ogh[F:2 * F], W_ogh[2 * F:2 * F + EDGE], W_ogh[2 * F + EDGE:], r(b_ogh),
        W_og, r(b_og),
    )
    return _tc_compute(gathered, nbr2, even_node, odd_node, weights)


# speedup vs baseline: 2.6764x; 2.6764x over previous
import functools

import jax
import jax.numpy as jnp
from jax import lax
from jax.experimental import pallas as pl
from jax.experimental.pallas import tpu as pltpu
from jax.experimental.pallas import tpu_sc as plsc

N = 10000
M = 32
F = 128
EDGE = 16
NE = N * M
B = 200
BM = B * M


def _sc_gather(table, idx):
    info = plsc.get_sparse_core_info()
    nc, ns = info.num_cores, info.num_subcores
    nw = nc * ns
    pw = NE // nw
    ch = 80
    nch = pw // ch

    mesh = plsc.VectorSubcoreMesh(core_axis_name="c", subcore_axis_name="s")

    @functools.partial(
        pl.kernel,
        out_type=jax.ShapeDtypeStruct((NE, 2 * F), jnp.float32),
        mesh=mesh,
        scratch_types=[
            pltpu.VMEM((ch,), jnp.int32),
            pltpu.VMEM((ch, 2 * F), jnp.float32),
            pltpu.SemaphoreType.DMA,
        ],
    )
    def gather_k(table_hbm, idx_hbm, out_hbm, idx_v, rows_v, sem):
        wid = lax.axis_index("s") * nc + lax.axis_index("c")
        base = wid * pw

        def body(c, carry):
            off = base + c * ch
            pltpu.sync_copy(idx_hbm.at[pl.ds(off, ch)], idx_v)
            pltpu.async_copy(table_hbm.at[idx_v], rows_v, sem).wait()
            pltpu.sync_copy(rows_v, out_hbm.at[pl.ds(off, ch)])
            return carry

        lax.fori_loop(0, nch, body, 0)

    return gather_k(table, idx)


def _softplus(x):
    return jnp.maximum(x, 0.0) + jnp.log1p(jnp.exp(-jnp.abs(x)))


def _sigmoid(x):
    return jax.nn.sigmoid(x)


def _dot(a, b):
    return lax.dot_general(a, b, (((1,), (0,)), ((), ())),
                           preferred_element_type=jnp.float32)


def _bc(x):
    return jnp.broadcast_to(x[:, None, :], (B, M, F)).reshape(BM, F)


def _tc_body(g_ref, nbr_ref, e_ref, o_ref,
             em1a, em1b, em1c, em1d, bem1,
             weg, beg, wem2, bem2,
             woej, boej, woei, boei, om1a, om1b,
             ogha, oghb, oghc, oghd, bogh, wog, bog,
             eout_ref, oout_ref):
    e_i = e_ref[...]
    o_i = o_ref[...]
    g = g_ref[...]
    ej = g[:, :F]
    oj = g[:, F:]
    nbr = nbr_ref[...]

    cross = _bc(o_i) * oj

    pre_e = (_bc(_dot(e_i, em1a[...])) + _dot(ej, em1b[...])
             + _dot(nbr, em1c[...]) + _dot(cross, em1d[...]) + bem1[...])
    h = _softplus(pre_e)
    gate = _sigmoid(_dot(h, weg[...]) + beg[...])
    msg = _softplus(_dot(h, wem2[...]) + bem2[...])
    even_agg = jnp.sum((gate * msg).reshape(B, M, F), axis=1)

    odd_ie = _bc(o_i) * (_dot(ej, woej[...]) + boej[...])
    odd_ei = _bc(_dot(e_i, woei[...]) + boei[...]) * oj
    odd_val = jnp.tanh(_dot(odd_ie, om1a[...]) + _dot(odd_ei, om1b[...]))

    pre_g = (_bc(_dot(e_i, ogha[...])) + _dot(ej, oghb[...])
             + _dot(nbr, oghc[...]) + _dot(cross, oghd[...]) + bogh[...])
    ogate = _sigmoid(_dot(_softplus(pre_g), wog[...]) + bog[...])
    odd_agg = jnp.sum((ogate * odd_val).reshape(B, M, F), axis=1)

    eout_ref[...] = e_i + even_agg
    oout_ref[...] = o_i + odd_agg


def _full(shape):
    return pl.BlockSpec(shape, lambda i: (0, 0))


def _tc_compute(gathered, nbr2, even_node, odd_node, weights):
    in_specs = [
        pl.BlockSpec((BM, 2 * F), lambda i: (i, 0)),
        pl.BlockSpec((BM, EDGE), lambda i: (i, 0)),
        pl.BlockSpec((B, F), lambda i: (i, 0)),
        pl.BlockSpec((B, F), lambda i: (i, 0)),
    ] + [_full(w.shape) for w in weights]
    out = pl.pallas_call(
        _tc_body,
        grid=(N // B,),
        in_specs=in_specs,
        out_specs=[pl.BlockSpec((B, F), lambda i: (i, 0))] * 2,
        out_shape=[jax.ShapeDtypeStruct((N, F), jnp.float32)] * 2,
    )(gathered, nbr2, even_node, odd_node, *weights)
    return tuple(out)


def kernel(even_node, odd_node, nbr_fea, nbr_fea_idx,
           W_em1, b_em1, W_eg, b_eg, W_em2, b_em2,
           W_oej, b_oej, W_oei, b_oei, W_om1,
           W_ogh, b_ogh, W_og, b_og):
    table = jnp.concatenate([even_node, odd_node], axis=1)
    idx = nbr_fea_idx.reshape(-1).astype(jnp.int32)
    gathered = _sc_gather(table, idx)
    nbr2 = nbr_fea.reshape(NE, EDGE)

    r = lambda b: b.reshape(1, F)
    weights = (
        W_em1[:F], W_em1[F:2 * F], W_em1[2 * F:2 * F + EDGE], W_em1[2 * F + EDGE:], r(b_em1),
        W_eg, r(b_eg), W_em2, r(b_em2),
        W_oej, r(b_oej), W_oei, r(b_oei), W_om1[:F], W_om1[F:],
        W_ogh[:F], W_ogh[F:2 * F], W_ogh[2 * F:2 * F + EDGE], W_ogh[2 * F + EDGE:], r(b_ogh),
        W_og, r(b_og),
    )
    return _tc_compute(gathered, nbr2, even_node, odd_node, weights)


# speedup vs baseline: 2.9867x; 1.1159x over previous
import functools

import jax
import jax.numpy as jnp
from jax import lax
from jax.experimental import pallas as pl
from jax.experimental.pallas import tpu as pltpu
from jax.experimental.pallas import tpu_sc as plsc

N = 10000
M = 32
F = 128
EDGE = 16
NE = N * M
B = 200
BM = B * M


def _sc_gather(table, idx):
    info = plsc.get_sparse_core_info()
    nc, ns = info.num_cores, info.num_subcores
    nw = nc * ns
    pw = NE // nw
    ch = 80
    nch = pw // ch

    mesh = plsc.VectorSubcoreMesh(core_axis_name="c", subcore_axis_name="s")

    @functools.partial(
        pl.kernel,
        out_type=jax.ShapeDtypeStruct((NE, F), jnp.float32),
        mesh=mesh,
        scratch_types=[
            pltpu.VMEM((ch,), jnp.int32),
            pltpu.VMEM((ch, F), jnp.float32),
            pltpu.SemaphoreType.DMA,
        ],
    )
    def gather_k(table_hbm, idx_hbm, out_hbm, idx_v, rows_v, sem):
        wid = lax.axis_index("s") * nc + lax.axis_index("c")
        base = wid * pw

        def body(c, carry):
            off = base + c * ch
            pltpu.sync_copy(idx_hbm.at[pl.ds(off, ch)], idx_v)
            pltpu.async_copy(table_hbm.at[idx_v], rows_v, sem).wait()
            pltpu.sync_copy(rows_v, out_hbm.at[pl.ds(off, ch)])
            return carry

        lax.fori_loop(0, nch, body, 0)

    return gather_k(table, idx)


def _softplus(x):
    return jnp.maximum(x, 0.0) + jnp.log1p(jnp.exp(-jnp.abs(x)))


def _sigmoid(x):
    return jax.nn.sigmoid(x)


def _dot(a, b):
    return lax.dot_general(a, b, (((1,), (0,)), ((), ())),
                           preferred_element_type=jnp.float32)


def _bc(x):
    return jnp.broadcast_to(x[:, None, :], (B, M, F)).reshape(BM, F)


def _tc_body(g_ref, nbr_ref, e_ref, o_ref,
             em1a, em1b, em1c, em1d, bem1,
             weg, beg, wem2, bem2,
             woej, boej, woei, boei, om1a, om1b,
             ogha, oghb, oghc, oghd, bogh, wog, bog,
             eout_ref, oout_ref):
    e_i = e_ref[...]
    o_i = o_ref[...]
    u = lax.bitcast_convert_type(g_ref[...], jnp.uint32)
    ej = lax.bitcast_convert_type(u << 16, jnp.float32)
    oj = lax.bitcast_convert_type(u & jnp.uint32(0xFFFF0000), jnp.float32)
    nbr = nbr_ref[...]

    cross = _bc(o_i) * oj

    pre_e = (_bc(_dot(e_i, em1a[...])) + _dot(ej, em1b[...])
             + _dot(nbr, em1c[...]) + _dot(cross, em1d[...]) + bem1[...])
    h = _softplus(pre_e)
    gate = _sigmoid(_dot(h, weg[...]) + beg[...])
    msg = _softplus(_dot(h, wem2[...]) + bem2[...])
    even_agg = jnp.sum((gate * msg).reshape(B, M, F), axis=1)

    odd_ie = _bc(o_i) * (_dot(ej, woej[...]) + boej[...])
    odd_ei = _bc(_dot(e_i, woei[...]) + boei[...]) * oj
    odd_val = jnp.tanh(_dot(odd_ie, om1a[...]) + _dot(odd_ei, om1b[...]))

    pre_g = (_bc(_dot(e_i, ogha[...])) + _dot(ej, oghb[...])
             + _dot(nbr, oghc[...]) + _dot(cross, oghd[...]) + bogh[...])
    ogate = _sigmoid(_dot(_softplus(pre_g), wog[...]) + bog[...])
    odd_agg = jnp.sum((ogate * odd_val).reshape(B, M, F), axis=1)

    eout_ref[...] = e_i + even_agg
    oout_ref[...] = o_i + odd_agg


def _full(shape):
    return pl.BlockSpec(shape, lambda i: (0, 0))


def _tc_compute(gathered, nbr2, even_node, odd_node, weights):
    in_specs = [
        pl.BlockSpec((BM, F), lambda i: (i, 0)),
        pl.BlockSpec((BM, EDGE), lambda i: (i, 0)),
        pl.BlockSpec((B, F), lambda i: (i, 0)),
        pl.BlockSpec((B, F), lambda i: (i, 0)),
    ] + [_full(w.shape) for w in weights]
    out = pl.pallas_call(
        _tc_body,
        grid=(N // B,),
        in_specs=in_specs,
        out_specs=[pl.BlockSpec((B, F), lambda i: (i, 0))] * 2,
        out_shape=[jax.ShapeDtypeStruct((N, F), jnp.float32)] * 2,
    )(gathered, nbr2, even_node, odd_node, *weights)
    return tuple(out)


def kernel(even_node, odd_node, nbr_fea, nbr_fea_idx,
           W_em1, b_em1, W_eg, b_eg, W_em2, b_em2,
           W_oej, b_oej, W_oei, b_oei, W_om1,
           W_ogh, b_ogh, W_og, b_og):
    ev = lax.bitcast_convert_type(even_node.astype(jnp.bfloat16), jnp.uint16).astype(jnp.uint32)
    od = lax.bitcast_convert_type(odd_node.astype(jnp.bfloat16), jnp.uint16).astype(jnp.uint32)
    table = lax.bitcast_convert_type((od << 16) | ev, jnp.float32)
    idx = nbr_fea_idx.reshape(-1).astype(jnp.int32)
    gathered = _sc_gather(table, idx)
    nbr2 = nbr_fea.reshape(NE, EDGE)

    r = lambda b: b.reshape(1, F)
    weights = (
        W_em1[:F], W_em1[F:2 * F], W_em1[2 * F:2 * F + EDGE], W_em1[2 * F + EDGE:], r(b_em1),
        W_eg, r(b_eg), W_em2, r(b_em2),
        W_oej, r(b_oej), W_oei, r(b_oei), W_om1[:F], W_om1[F:],
        W_ogh[:F], W_ogh[F:2 * F], W_ogh[2 * F:2 * F + EDGE], W_ogh[2 * F + EDGE:], r(b_ogh),
        W_og, r(b_og),
    )
    return _tc_compute(gathered, nbr2, even_node, odd_node, weights)


# speedup vs baseline: 3.3947x; 1.1366x over previous
import functools

import jax
import jax.numpy as jnp
from jax import lax
from jax.experimental import pallas as pl
from jax.experimental.pallas import tpu as pltpu
from jax.experimental.pallas import tpu_sc as plsc

N = 10000
M = 32
F = 128
EDGE = 16
NE = N * M
B = 200
BM = B * M


def _sc_gather(table, idx, ne):
    info = plsc.get_sparse_core_info()
    nc, ns = info.num_cores, info.num_subcores
    nw = nc * ns
    pw = ne // nw
    ch = 80
    nch = pw // ch

    mesh = plsc.VectorSubcoreMesh(core_axis_name="c", subcore_axis_name="s")

    @functools.partial(
        pl.kernel,
        out_type=jax.ShapeDtypeStruct((ne, F), jnp.float32),
        mesh=mesh,
        scratch_types=[
            pltpu.VMEM((ch,), jnp.int32),
            pltpu.VMEM((ch, F), jnp.float32),
            pltpu.SemaphoreType.DMA,
        ],
    )
    def gather_k(table_hbm, idx_hbm, out_hbm, idx_v, rows_v, sem):
        wid = lax.axis_index("s") * nc + lax.axis_index("c")
        base = wid * pw

        def body(c, carry):
            off = base + c * ch
            pltpu.sync_copy(idx_hbm.at[pl.ds(off, ch)], idx_v)
            pltpu.async_copy(table_hbm.at[idx_v], rows_v, sem).wait()
            pltpu.sync_copy(rows_v, out_hbm.at[pl.ds(off, ch)])
            return carry

        lax.fori_loop(0, nch, body, 0)

    return gather_k(table, idx)


def _softplus(x):
    return jnp.maximum(x, 0.0) + jnp.log1p(jnp.exp(-jnp.abs(x)))


def _sigmoid(x):
    return jax.nn.sigmoid(x)


def _dot(a, b):
    return lax.dot_general(a, b, (((1,), (0,)), ((), ())),
                           preferred_element_type=jnp.float32)


def _bc(x):
    return jnp.broadcast_to(x[:, None, :], (B, M, F)).reshape(BM, F)


def _tc_body(g_ref, nbr_ref, e_ref, o_ref,
             em1a, em1b, em1c, em1d, bem1,
             weg, beg, wem2, bem2,
             woej, boej, woei, boei, om1a, om1b,
             ogha, oghb, oghc, oghd, bogh, wog, bog,
             eout_ref, oout_ref):
    e_i = e_ref[...]
    o_i = o_ref[...]
    u = lax.bitcast_convert_type(g_ref[...], jnp.uint32)
    ej = lax.bitcast_convert_type(u << 16, jnp.float32)
    oj = lax.bitcast_convert_type(u & jnp.uint32(0xFFFF0000), jnp.float32)
    nbr = nbr_ref[...]

    cross = _bc(o_i) * oj

    pre_e = (_bc(_dot(e_i, em1a[...])) + _dot(ej, em1b[...])
             + _dot(nbr, em1c[...]) + _dot(cross, em1d[...]) + bem1[...])
    h = _softplus(pre_e)
    gate = _sigmoid(_dot(h, weg[...]) + beg[...])
    msg = _softplus(_dot(h, wem2[...]) + bem2[...])
    even_agg = jnp.sum((gate * msg).reshape(B, M, F), axis=1)

    odd_ie = _bc(o_i) * (_dot(ej, woej[...]) + boej[...])
    odd_ei = _bc(_dot(e_i, woei[...]) + boei[...]) * oj
    odd_val = jnp.tanh(_dot(odd_ie, om1a[...]) + _dot(odd_ei, om1b[...]))

    pre_g = (_bc(_dot(e_i, ogha[...])) + _dot(ej, oghb[...])
             + _dot(nbr, oghc[...]) + _dot(cross, oghd[...]) + bogh[...])
    ogate = _sigmoid(_dot(_softplus(pre_g), wog[...]) + bog[...])
    odd_agg = jnp.sum((ogate * odd_val).reshape(B, M, F), axis=1)

    eout_ref[...] = e_i + even_agg
    oout_ref[...] = o_i + odd_agg


def _full(shape):
    return pl.BlockSpec(shape, lambda i: (0, 0))


def _tc_compute(gathered, nbr2, even_node, odd_node, weights):
    nn = even_node.shape[0]
    in_specs = [
        pl.BlockSpec((BM, F), lambda i: (i, 0)),
        pl.BlockSpec((BM, EDGE), lambda i: (i, 0)),
        pl.BlockSpec((B, F), lambda i: (i, 0)),
        pl.BlockSpec((B, F), lambda i: (i, 0)),
    ] + [_full(w.shape) for w in weights]
    out = pl.pallas_call(
        _tc_body,
        grid=(nn // B,),
        in_specs=in_specs,
        out_specs=[pl.BlockSpec((B, F), lambda i: (i, 0))] * 2,
        out_shape=[jax.ShapeDtypeStruct((nn, F), jnp.float32)] * 2,
    )(gathered, nbr2, even_node, odd_node, *weights)
    return tuple(out)


def kernel(even_node, odd_node, nbr_fea, nbr_fea_idx,
           W_em1, b_em1, W_eg, b_eg, W_em2, b_em2,
           W_oej, b_oej, W_oei, b_oei, W_om1,
           W_ogh, b_ogh, W_og, b_og):
    ev = lax.bitcast_convert_type(even_node.astype(jnp.bfloat16), jnp.uint16).astype(jnp.uint32)
    od = lax.bitcast_convert_type(odd_node.astype(jnp.bfloat16), jnp.uint16).astype(jnp.uint32)
    table = lax.bitcast_convert_type((od << 16) | ev, jnp.float32)
    idx = nbr_fea_idx.reshape(-1).astype(jnp.int32)
    nbr2 = nbr_fea.reshape(NE, EDGE)

    r = lambda b: b.reshape(1, F)
    weights = (
        W_em1[:F], W_em1[F:2 * F], W_em1[2 * F:2 * F + EDGE], W_em1[2 * F + EDGE:], r(b_em1),
        W_eg, r(b_eg), W_em2, r(b_em2),
        W_oej, r(b_oej), W_oei, r(b_oei), W_om1[:F], W_om1[F:],
        W_ogh[:F], W_ogh[F:2 * F], W_ogh[2 * F:2 * F + EDGE], W_ogh[2 * F + EDGE:], r(b_ogh),
        W_og, r(b_og),
    )

    K = 5
    ns_ = N // K
    nes = ns_ * M
    evens, odds = [], []
    for k in range(K):
        g_k = _sc_gather(table, lax.slice_in_dim(idx, k * nes, (k + 1) * nes), nes)
        e_k, o_k = _tc_compute(
            g_k,
            lax.slice_in_dim(nbr2, k * nes, (k + 1) * nes),
            lax.slice_in_dim(even_node, k * ns_, (k + 1) * ns_),
            lax.slice_in_dim(odd_node, k * ns_, (k + 1) * ns_),
            weights,
        )
        evens.append(e_k)
        odds.append(o_k)
    return jnp.concatenate(evens, axis=0), jnp.concatenate(odds, axis=0)


# speedup vs baseline: 3.7621x; 1.1082x over previous
import functools

import jax
import jax.numpy as jnp
from jax import lax
from jax.experimental import pallas as pl
from jax.experimental.pallas import tpu as pltpu
from jax.experimental.pallas import tpu_sc as plsc

N = 10000
M = 32
F = 128
EDGE = 16
NE = N * M
B = 200
BM = B * M


def _sc_gather(table, idx, ne):
    info = plsc.get_sparse_core_info()
    nc, ns = info.num_cores, info.num_subcores
    nw = nc * ns
    pw = ne // nw
    ch = 80
    nch = pw // ch

    mesh = plsc.VectorSubcoreMesh(core_axis_name="c", subcore_axis_name="s")

    @functools.partial(
        pl.kernel,
        out_type=jax.ShapeDtypeStruct((ne, F), jnp.float32),
        mesh=mesh,
        scratch_types=[
            pltpu.VMEM((ch,), jnp.int32),
            pltpu.VMEM((ch, F), jnp.float32),
            pltpu.SemaphoreType.DMA,
        ],
    )
    def gather_k(table_hbm, idx_hbm, out_hbm, idx_v, rows_v, sem):
        wid = lax.axis_index("s") * nc + lax.axis_index("c")
        base = wid * pw

        def body(c, carry):
            off = base + c * ch
            pltpu.sync_copy(idx_hbm.at[pl.ds(off, ch)], idx_v)
            pltpu.async_copy(table_hbm.at[idx_v], rows_v, sem).wait()
            pltpu.sync_copy(rows_v, out_hbm.at[pl.ds(off, ch)])
            return carry

        lax.fori_loop(0, nch, body, 0)

    return gather_k(table, idx)


def _softplus(x):
    return jnp.maximum(x, 0.0) + jnp.log(1.0 + jnp.exp(-jnp.abs(x)))


def _sigmoid(x):
    return jax.nn.sigmoid(x)


def _dot(a, b):
    return lax.dot_general(a, b, (((1,), (0,)), ((), ())),
                           preferred_element_type=jnp.float32)


def _bc(x):
    return jnp.broadcast_to(x[:, None, :], (B, M, F)).reshape(BM, F)


def _bc2(x):
    return jnp.broadcast_to(x[:, None, :], (B, M, 2 * F)).reshape(BM, 2 * F)


def _tc_body(g_ref, nbr_ref, e_ref, o_ref,
             w_ec, w_n2, w_i2, bias2,
             w_gm, bias_gm,
             woej, boej, woei, boei, om1a, om1b,
             wog, bog,
             eout_ref, oout_ref):
    e_i = e_ref[...]
    o_i = o_ref[...]
    u = lax.bitcast_convert_type(g_ref[...], jnp.uint32)
    ej = lax.bitcast_convert_type(u << 16, jnp.float32)
    oj = lax.bitcast_convert_type(u & jnp.uint32(0xFFFF0000), jnp.float32)
    nbr = nbr_ref[...].reshape(BM, EDGE)

    cross = _bc(o_i) * oj
    ec = jnp.concatenate([ej, cross], axis=1)

    pre = (_dot(ec, w_ec[...]) + _dot(nbr, w_n2[...])
           + _bc2(_dot(e_i, w_i2[...])) + bias2[...])
    s = _softplus(pre)
    h = s[:, :F]
    hg = s[:, F:]

    gm = _dot(h, w_gm[...]) + bias_gm[...]
    gate = _sigmoid(gm[:, :F])
    msg = _softplus(gm[:, F:])
    even_agg = jnp.sum((gate * msg).reshape(B, M, F), axis=1)

    odd_ie = _bc(o_i) * (_dot(ej, woej[...]) + boej[...])
    odd_ei = _bc(_dot(e_i, woei[...]) + boei[...]) * oj
    odd_val = jnp.tanh(_dot(odd_ie, om1a[...]) + _dot(odd_ei, om1b[...]))

    ogate = _sigmoid(_dot(hg, wog[...]) + bog[...])
    odd_agg = jnp.sum((ogate * odd_val).reshape(B, M, F), axis=1)

    eout_ref[...] = e_i + even_agg
    oout_ref[...] = o_i + odd_agg


def _full(shape):
    return pl.BlockSpec(shape, lambda i: (0, 0))


def _tc_compute(gathered, nbr3, even_node, odd_node, weights):
    nn = even_node.shape[0]
    in_specs = [
        pl.BlockSpec((BM, F), lambda i: (i, 0)),
        pl.BlockSpec((B, M, EDGE), lambda i: (i, 0, 0)),
        pl.BlockSpec((B, F), lambda i: (i, 0)),
        pl.BlockSpec((B, F), lambda i: (i, 0)),
    ] + [_full(w.shape) for w in weights]
    out = pl.pallas_call(
        _tc_body,
        grid=(nn // B,),
        in_specs=in_specs,
        out_specs=[pl.BlockSpec((B, F), lambda i: (i, 0))] * 2,
        out_shape=[jax.ShapeDtypeStruct((nn, F), jnp.float32)] * 2,
    )(gathered, nbr3, even_node, odd_node, *weights)
    return tuple(out)


def kernel(even_node, odd_node, nbr_fea, nbr_fea_idx,
           W_em1, b_em1, W_eg, b_eg, W_em2, b_em2,
           W_oej, b_oej, W_oei, b_oei, W_om1,
           W_ogh, b_ogh, W_og, b_og):
    ev = lax.bitcast_convert_type(even_node.astype(jnp.bfloat16), jnp.uint16).astype(jnp.uint32)
    od = lax.bitcast_convert_type(odd_node.astype(jnp.bfloat16), jnp.uint16).astype(jnp.uint32)
    table = lax.bitcast_convert_type((od << 16) | ev, jnp.float32)
    idx = nbr_fea_idx.reshape(-1).astype(jnp.int32)

    r = lambda b: b.reshape(1, F)
    cat = lambda a, b2: jnp.concatenate([a, b2], axis=1)
    w_ec = jnp.concatenate(
        [cat(W_em1[F:2 * F], W_ogh[F:2 * F]),
         cat(W_em1[2 * F + EDGE:], W_ogh[2 * F + EDGE:])], axis=0)
    w_n2 = cat(W_em1[2 * F:2 * F + EDGE], W_ogh[2 * F:2 * F + EDGE])
    w_i2 = cat(W_em1[:F], W_ogh[:F])
    bias2 = cat(r(b_em1), r(b_ogh))
    w_gm = cat(W_eg, W_em2)
    bias_gm = cat(r(b_eg), r(b_em2))
    weights = (
        w_ec, w_n2, w_i2, bias2, w_gm, bias_gm,
        W_oej, r(b_oej), W_oei, r(b_oei), W_om1[:F], W_om1[F:],
        W_og, r(b_og),
    )

    K = 5
    ns_ = N // K
    nes = ns_ * M
    evens, odds = [], []
    for k in range(K):
        g_k = _sc_gather(table, lax.slice_in_dim(idx, k * nes, (k + 1) * nes), nes)
        e_k, o_k = _tc_compute(
            g_k,
            lax.slice_in_dim(nbr_fea, k * ns_, (k + 1) * ns_),
            lax.slice_in_dim(even_node, k * ns_, (k + 1) * ns_),
            lax.slice_in_dim(odd_node, k * ns_, (k + 1) * ns_),
            weights,
        )
        evens.append(e_k)
        odds.append(o_k)
    return jnp.concatenate(evens, axis=0), jnp.concatenate(odds, axis=0)


# speedup vs baseline: 3.7637x; 1.0004x over previous
import functools

import jax
import jax.numpy as jnp
from jax import lax
from jax.experimental import pallas as pl
from jax.experimental.pallas import tpu as pltpu
from jax.experimental.pallas import tpu_sc as plsc

N = 10000
M = 32
F = 128
EDGE = 16
NE = N * M
B = 200
BM = B * M


def _sc_gather(table, idx, ne):
    info = plsc.get_sparse_core_info()
    nc, ns = info.num_cores, info.num_subcores
    nw = nc * ns
    pw = ne // nw
    ch = 80
    nch = pw // ch

    mesh = plsc.VectorSubcoreMesh(core_axis_name="c", subcore_axis_name="s")

    @functools.partial(
        pl.kernel,
        out_type=jax.ShapeDtypeStruct((ne, F), jnp.float32),
        mesh=mesh,
        scratch_types=[
            pltpu.VMEM((ch,), jnp.int32),
            pltpu.VMEM((ch, F), jnp.float32),
            pltpu.SemaphoreType.DMA,
        ],
        compiler_params=pltpu.CompilerParams(use_tc_tiling_on_sc=True),
    )
    def gather_k(table_hbm, idx_hbm, out_hbm, idx_v, rows_v, sem):
        wid = lax.axis_index("s") * nc + lax.axis_index("c")
        base = wid * pw

        def body(c, carry):
            off = base + c * ch
            pltpu.sync_copy(idx_hbm.at[pl.ds(off, ch)], idx_v)
            pltpu.async_copy(table_hbm.at[idx_v], rows_v, sem).wait()
            pltpu.sync_copy(rows_v, out_hbm.at[pl.ds(off, ch)])
            return carry

        lax.fori_loop(0, nch, body, 0)

    return gather_k(table, idx)


def _softplus(x):
    return jnp.maximum(x, 0.0) + jnp.log(1.0 + jnp.exp(-jnp.abs(x)))


def _sigmoid(x):
    return jax.nn.sigmoid(x)


def _dot(a, b):
    return lax.dot_general(a, b, (((1,), (0,)), ((), ())),
                           preferred_element_type=jnp.float32)


def _bc(x):
    return jnp.broadcast_to(x[:, None, :], (B, M, F)).reshape(BM, F)


def _bc2(x):
    return jnp.broadcast_to(x[:, None, :], (B, M, 2 * F)).reshape(BM, 2 * F)


def _tc_body(g_ref, nbr_ref, e_ref, o_ref,
             w_ec, w_n2, w_i2, bias2,
             w_gm, bias_gm,
             woej, boej, woei, boei, om1a, om1b,
             wog, bog,
             eout_ref, oout_ref):
    e_i = e_ref[...]
    o_i = o_ref[...]
    u = lax.bitcast_convert_type(g_ref[...], jnp.uint32)
    ej = lax.bitcast_convert_type(u << 16, jnp.float32)
    oj = lax.bitcast_convert_type(u & jnp.uint32(0xFFFF0000), jnp.float32)
    nbr = nbr_ref[...].reshape(BM, EDGE)

    cross = _bc(o_i) * oj
    ec = jnp.concatenate([ej, cross], axis=1)

    pre = (_dot(ec, w_ec[...]) + _dot(nbr, w_n2[...])
           + _bc2(_dot(e_i, w_i2[...])) + bias2[...])
    s = _softplus(pre)
    h = s[:, :F]
    hg = s[:, F:]

    gm = _dot(h, w_gm[...]) + bias_gm[...]
    gate = _sigmoid(gm[:, :F])
    msg = _softplus(gm[:, F:])
    even_agg = jnp.sum((gate * msg).reshape(B, M, F), axis=1)

    odd_ie = _bc(o_i) * (_dot(ej, woej[...]) + boej[...])
    odd_ei = _bc(_dot(e_i, woei[...]) + boei[...]) * oj
    odd_val = jnp.tanh(_dot(odd_ie, om1a[...]) + _dot(odd_ei, om1b[...]))

    ogate = _sigmoid(_dot(hg, wog[...]) + bog[...])
    odd_agg = jnp.sum((ogate * odd_val).reshape(B, M, F), axis=1)

    eout_ref[...] = e_i + even_agg
    oout_ref[...] = o_i + odd_agg


def _full(shape):
    return pl.BlockSpec(shape, lambda i: (0, 0))


def _tc_compute(gathered, nbr3, even_node, odd_node, weights):
    nn = even_node.shape[0]
    in_specs = [
        pl.BlockSpec((BM, F), lambda i: (i, 0)),
        pl.BlockSpec((B, M, EDGE), lambda i: (i, 0, 0)),
        pl.BlockSpec((B, F), lambda i: (i, 0)),
        pl.BlockSpec((B, F), lambda i: (i, 0)),
    ] + [_full(w.shape) for w in weights]
    out = pl.pallas_call(
        _tc_body,
        grid=(nn // B,),
        in_specs=in_specs,
        out_specs=[pl.BlockSpec((B, F), lambda i: (i, 0))] * 2,
        out_shape=[jax.ShapeDtypeStruct((nn, F), jnp.float32)] * 2,
    )(gathered, nbr3, even_node, odd_node, *weights)
    return tuple(out)


def kernel(even_node, odd_node, nbr_fea, nbr_fea_idx,
           W_em1, b_em1, W_eg, b_eg, W_em2, b_em2,
           W_oej, b_oej, W_oei, b_oei, W_om1,
           W_ogh, b_ogh, W_og, b_og):
    ev = lax.bitcast_convert_type(even_node.astype(jnp.bfloat16), jnp.uint16).astype(jnp.uint32)
    od = lax.bitcast_convert_type(odd_node.astype(jnp.bfloat16), jnp.uint16).astype(jnp.uint32)
    table = lax.bitcast_convert_type((od << 16) | ev, jnp.float32)
    idx = nbr_fea_idx.reshape(-1).astype(jnp.int32)

    r = lambda b: b.reshape(1, F)
    cat = lambda a, b2: jnp.concatenate([a, b2], axis=1)
    w_ec = jnp.concatenate(
        [cat(W_em1[F:2 * F], W_ogh[F:2 * F]),
         cat(W_em1[2 * F + EDGE:], W_ogh[2 * F + EDGE:])], axis=0)
    w_n2 = cat(W_em1[2 * F:2 * F + EDGE], W_ogh[2 * F:2 * F + EDGE])
    w_i2 = cat(W_em1[:F], W_ogh[:F])
    bias2 = cat(r(b_em1), r(b_ogh))
    w_gm = cat(W_eg, W_em2)
    bias_gm = cat(r(b_eg), r(b_em2))
    weights = (
        w_ec, w_n2, w_i2, bias2, w_gm, bias_gm,
        W_oej, r(b_oej), W_oei, r(b_oei), W_om1[:F], W_om1[F:],
        W_og, r(b_og),
    )

    K = 5
    ns_ = N // K
    nes = ns_ * M
    evens, odds = [], []
    for k in range(K):
        g_k = _sc_gather(table, lax.slice_in_dim(idx, k * nes, (k + 1) * nes), nes)
        e_k, o_k = _tc_compute(
            g_k,
            lax.slice_in_dim(nbr_fea, k * ns_, (k + 1) * ns_),
            lax.slice_in_dim(even_node, k * ns_, (k + 1) * ns_),
            lax.slice_in_dim(odd_node, k * ns_, (k + 1) * ns_),
            weights,
        )
        evens.append(e_k)
        odds.append(o_k)
    return jnp.concatenate(evens, axis=0), jnp.concatenate(odds, axis=0)


# speedup vs baseline: 4.1474x; 1.1020x over previous
import functools

import jax
import jax.numpy as jnp
from jax import lax
from jax.experimental import pallas as pl
from jax.experimental.pallas import tpu as pltpu
from jax.experimental.pallas import tpu_sc as plsc

N = 10000
M = 32
F = 128
EDGE = 16
NE = N * M
B = 200
BM = B * M


def _sc_gather(table, idx, ne, e0):
    info = plsc.get_sparse_core_info()
    nc, ns = info.num_cores, info.num_subcores
    nw = nc * ns
    pw = ne // nw
    ch = 80
    nch = pw // ch

    mesh = plsc.VectorSubcoreMesh(core_axis_name="c", subcore_axis_name="s")

    @functools.partial(
        pl.kernel,
        out_type=jax.ShapeDtypeStruct((ne, F), jnp.float32),
        mesh=mesh,
        scratch_types=[
            pltpu.VMEM((ch,), jnp.int32),
            pltpu.VMEM((ch, F), jnp.float32),
            pltpu.SemaphoreType.DMA,
        ],
        compiler_params=pltpu.CompilerParams(use_tc_tiling_on_sc=True),
    )
    def gather_k(table_hbm, idx_hbm, out_hbm, idx_v, rows_v, sem):
        wid = lax.axis_index("s") * nc + lax.axis_index("c")
        base = wid * pw

        def body(c, carry):
            off = base + c * ch
            pltpu.sync_copy(idx_hbm.at[pl.ds(e0 + off, ch)], idx_v)
            pltpu.async_copy(table_hbm.at[idx_v], rows_v, sem).wait()
            pltpu.sync_copy(rows_v, out_hbm.at[pl.ds(off, ch)])
            return carry

        lax.fori_loop(0, nch, body, 0)

    return gather_k(table, idx)


def _softplus(x):
    return jnp.maximum(x, 0.0) + jnp.log(1.0 + jnp.exp(-jnp.abs(x)))


def _sigmoid(x):
    return jax.nn.sigmoid(x)


def _dot(a, b):
    return lax.dot_general(a, b, (((1,), (0,)), ((), ())),
                           preferred_element_type=jnp.float32)


def _bc(x):
    return jnp.broadcast_to(x[:, None, :], (B, M, F)).reshape(BM, F)


def _bc2(x):
    return jnp.broadcast_to(x[:, None, :], (B, M, 2 * F)).reshape(BM, 2 * F)


def _tc_body(g_ref, nbr_ref, e_ref, o_ref,
             w_ec, w_n2, w_i2, bias2,
             w_gm, bias_gm,
             woej, boej, woei, boei, om1a, om1b,
             wog, bog,
             eout_ref, oout_ref):
    e_i = e_ref[...]
    o_i = o_ref[...]
    u = lax.bitcast_convert_type(g_ref[...], jnp.uint32)
    ej = lax.bitcast_convert_type(u << 16, jnp.float32)
    oj = lax.bitcast_convert_type(u & jnp.uint32(0xFFFF0000), jnp.float32)
    nbr = nbr_ref[...].reshape(BM, EDGE)

    cross = _bc(o_i) * oj
    ec = jnp.concatenate([ej, cross], axis=1)

    pre = (_dot(ec, w_ec[...]) + _dot(nbr, w_n2[...])
           + _bc2(_dot(e_i, w_i2[...])) + bias2[...])
    s = _softplus(pre)
    h = s[:, :F]
    hg = s[:, F:]

    gm = _dot(h, w_gm[...]) + bias_gm[...]
    gate = _sigmoid(gm[:, :F])
    msg = _softplus(gm[:, F:])
    even_agg = jnp.sum((gate * msg).reshape(B, M, F), axis=1)

    odd_ie = _bc(o_i) * (_dot(ej, woej[...]) + boej[...])
    odd_ei = _bc(_dot(e_i, woei[...]) + boei[...]) * oj
    odd_val = jnp.tanh(_dot(odd_ie, om1a[...]) + _dot(odd_ei, om1b[...]))

    ogate = _sigmoid(_dot(hg, wog[...]) + bog[...])
    odd_agg = jnp.sum((ogate * odd_val).reshape(B, M, F), axis=1)

    eout_ref[...] = e_i + even_agg
    oout_ref[...] = o_i + odd_agg


def _full(shape):
    return pl.BlockSpec(shape, lambda i: (0, 0))


def _tc_compute(gathered, nbr3, even_node, odd_node, weights, b0, nb):
    in_specs = [
        pl.BlockSpec((BM, F), lambda i: (i, 0)),
        pl.BlockSpec((B, M, EDGE), lambda i: (i + b0, 0, 0)),
        pl.BlockSpec((B, F), lambda i: (i + b0, 0)),
        pl.BlockSpec((B, F), lambda i: (i + b0, 0)),
    ] + [_full(w.shape) for w in weights]
    out = pl.pallas_call(
        _tc_body,
        grid=(nb,),
        in_specs=in_specs,
        out_specs=[pl.BlockSpec((B, F), lambda i: (i, 0))] * 2,
        out_shape=[jax.ShapeDtypeStruct((nb * B, F), jnp.float32)] * 2,
    )(gathered, nbr3, even_node, odd_node, *weights)
    return tuple(out)


def kernel(even_node, odd_node, nbr_fea, nbr_fea_idx,
           W_em1, b_em1, W_eg, b_eg, W_em2, b_em2,
           W_oej, b_oej, W_oei, b_oei, W_om1,
           W_ogh, b_ogh, W_og, b_og):
    ev = lax.bitcast_convert_type(even_node.astype(jnp.bfloat16), jnp.uint16).astype(jnp.uint32)
    od = lax.bitcast_convert_type(odd_node.astype(jnp.bfloat16), jnp.uint16).astype(jnp.uint32)
    table = lax.bitcast_convert_type((od << 16) | ev, jnp.float32)
    idx = nbr_fea_idx.reshape(-1).astype(jnp.int32)

    r = lambda b: b.reshape(1, F)
    cat = lambda a, b2: jnp.concatenate([a, b2], axis=1)
    w_ec = jnp.concatenate(
        [cat(W_em1[F:2 * F], W_ogh[F:2 * F]),
         cat(W_em1[2 * F + EDGE:], W_ogh[2 * F + EDGE:])], axis=0)
    w_n2 = cat(W_em1[2 * F:2 * F + EDGE], W_ogh[2 * F:2 * F + EDGE])
    w_i2 = cat(W_em1[:F], W_ogh[:F])
    bias2 = cat(r(b_em1), r(b_ogh))
    w_gm = cat(W_eg, W_em2)
    bias_gm = cat(r(b_eg), r(b_em2))
    weights = (
        w_ec, w_n2, w_i2, bias2, w_gm, bias_gm,
        W_oej, r(b_oej), W_oei, r(b_oei), W_om1[:F], W_om1[F:],
        W_og, r(b_og),
    )

    slices = (800, 1600, 2400, 2400, 2800)
    evens, odds = [], []
    n0 = 0
    for ns_ in slices:
        g_k = _sc_gather(table, idx, ns_ * M, n0 * M)
        e_k, o_k = _tc_compute(g_k, nbr_fea, even_node, odd_node, weights,
                               n0 // B, ns_ // B)
        evens.append(e_k)
        odds.append(o_k)
        n0 += ns_
    return jnp.concatenate(evens, axis=0), jnp.concatenate(odds, axis=0)
